# baseline (speedup 1.0000x reference)
import jax
import jax.numpy as jnp
from jax import lax
from jax.experimental import pallas as pl
from jax.experimental.pallas import tpu as pltpu

_TM = 512
_RP = 128


def _fused_lora_kernel(x_ref, w_hbm, a_ref, b_ref, bias_ref, o_ref,
                       wcat_ref, sem):
    r, K = a_ref.shape
    N = b_ref.shape[0]

    @pl.when(pl.program_id(0) == 0)
    def _():
        cp = pltpu.make_async_copy(w_hbm, wcat_ref.at[:N, :], sem)
        cp.start()
        wcat_ref[N:N + r, :] = a_ref[...]
        wcat_ref[N + r:, :] = jnp.zeros((_RP - r, K), jnp.float32)
        cp.wait()

    nt = (((1,), (1,)), ((), ()))
    big = lax.dot_general(x_ref[...], wcat_ref[...], nt,
                          preferred_element_type=jnp.float32)

    lora = lax.dot_general(big[:, N:N + r], b_ref[...], nt,
                           preferred_element_type=jnp.float32)
    o_ref[...] = big[:, :N] + lora + bias_ref[...]


def kernel(x, w, bias, A, B):
    lead = x.shape[:-1]
    K = x.shape[-1]
    N = w.shape[0]
    r = A.shape[0]

    x2 = x.reshape(-1, K)
    M = x2.shape[0]
    tm = min(_TM, M)
    bias2 = bias.reshape(1, N)

    y = pl.pallas_call(
        _fused_lora_kernel,
        out_shape=jax.ShapeDtypeStruct((M, N), jnp.float32),
        grid=(M // tm,),
        in_specs=[
            pl.BlockSpec((tm, K), lambda i: (i, 0)),
            pl.BlockSpec(memory_space=pl.ANY),
            pl.BlockSpec((r, K), lambda i: (0, 0)),
            pl.BlockSpec((N, r), lambda i: (0, 0)),
            pl.BlockSpec((1, N), lambda i: (0, 0)),
        ],
        out_specs=pl.BlockSpec((tm, N), lambda i: (i, 0)),
        scratch_shapes=[
            pltpu.VMEM((N + _RP, K), jnp.float32),
            pltpu.SemaphoreType.DMA,
        ],
        compiler_params=pltpu.CompilerParams(
            dimension_semantics=("arbitrary",),
            vmem_limit_bytes=60 * 1024 * 1024,
        ),
        cost_estimate=pl.CostEstimate(
            flops=2 * M * K * (N + _RP) + 2 * M * r * N,
            transcendentals=0,
            bytes_accessed=(M * K + M * N + N * K) * 4,
        ),
    )(x2, w, A, B, bias2)
    return y.reshape(*lead, N)
